# baseline (speedup 1.0000x reference)
import functools

import jax
import jax.numpy as jnp
from jax import lax
from jax.experimental import pallas as pl
from jax.experimental.pallas import tpu as pltpu
from jax.experimental.pallas import tpu_sc as plsc

NUM_N = 8192
IN_SZ = 2048
BATCH = 1024

SC_CORES = 2
SC_SUBCORES = 16
NW = SC_CORES * SC_SUBCORES
LANES = 16
B_W = BATCH // NW
BP = B_W + 1
NC_OUT = 128
NCP = NC_OUT + 8
T_STEPS = NUM_N // NC_OUT
CCH = 256


def _coef_body(wt_ref, it_ref, pk_ref):
    wv = wt_ref[...]
    m = jnp.max(wv, axis=0, keepdims=True)
    e = jnp.exp(wv - m)
    p = e / jnp.sum(e, axis=0, keepdims=True)
    c0 = p[8] + p[9] + p[10] + p[11] + p[12] + p[13] + p[14] + p[15]
    c1 = p[2] + p[3] + p[6] + p[7] - p[8] - p[9] - p[12] - p[13]
    c2 = p[4] + p[5] + p[6] + p[7] - p[8] - p[9] - p[10] - p[11]
    c3 = (p[1] - p[2] - p[4] - 2.0 * p[6] - p[7] + p[8] + 2.0 * p[9]
          + p[11] + p[13] - p[14])
    i12 = it_ref[0, :] | (it_ref[1, :] << 16)
    pk_ref[pl.ds(0, NUM_N)] = lax.bitcast_convert_type(i12, jnp.float32)
    pk_ref[pl.ds(NUM_N, NUM_N)] = c0
    pk_ref[pl.ds(2 * NUM_N, NUM_N)] = c1
    pk_ref[pl.ds(3 * NUM_N, NUM_N)] = c2
    pk_ref[pl.ds(4 * NUM_N, NUM_N)] = c3


def _packed_side(w, conn_indices):
    return pl.pallas_call(
        _coef_body,
        out_shape=jax.ShapeDtypeStruct((5 * NUM_N,), jnp.float32),
    )(w.T, conn_indices.T)


def _sc_body(x_hbm, pk_hbm, out_hbm, xt, xr, cb, ob0, ob1,
             sem_x, sem_pk, sem_o0, sem_o1):
    wid = lax.axis_index("s") * SC_CORES + lax.axis_index("c")
    b0 = wid * B_W
    cp = pltpu.async_copy(pk_hbm, cb, sem_pk)
    iota = lax.iota(jnp.int32, LANES)
    i33 = iota * BP
    rows0 = iota
    rows1 = iota + LANES

    for cc in range(IN_SZ // CCH):
        c0 = cc * CCH
        for r in range(B_W):
            pltpu.async_copy(x_hbm.at[b0 + r, pl.ds(c0, CCH)],
                             xr.at[pl.ds(r * CCH, CCH)], sem_x)
        for r in range(B_W):
            pltpu.make_async_copy(x_hbm.at[b0 + r, pl.ds(c0, CCH)],
                                  xr.at[pl.ds(r * CCH, CCH)], sem_x).wait()

        @plsc.parallel_loop(0, B_W, 1, unroll=2)
        def _tr(r):
            rb = r * CCH
            cbase = c0 * BP + r
            for g in range(CCH // LANES):
                v = xr[pl.ds(rb + g * LANES, LANES)]
                plsc.store_scatter(xt, [i33 + (cbase + g * LANES * BP)], v)

    cp.wait()

    def _chunk(t, obuf, osem, first):
        n0 = t * NC_OUT
        if not first:
            pltpu.make_async_copy(
                obuf,
                out_hbm.at[pl.ds(b0, B_W), pl.ds(n0, NC_OUT)], osem).wait()

        @pl.loop(0, NC_OUT // LANES)
        def _group(g):
            gb = n0 + g * LANES
            i12 = plsc.bitcast(cb[pl.ds(gb, LANES)], jnp.int32)
            c0g = cb[pl.ds(NUM_N + gb, LANES)]
            c1g = cb[pl.ds(2 * NUM_N + gb, LANES)]
            c2g = cb[pl.ds(3 * NUM_N + gb, LANES)]
            c3g = cb[pl.ds(4 * NUM_N + gb, LANES)]
            gol = g * LANES
            for k in range(LANES):
                s12 = i12[k]
                si1 = (s12 & 0xFFFF) * BP
                si2 = lax.shift_right_logical(s12, 16) * BP
                c0n = c0g[k]
                c1n = c1g[k]
                c2n = c2g[k]
                c3n = c3g[k]
                ncol = jnp.full((LANES,), gol + k, jnp.int32)
                for h, rows in ((0, rows0), (1, rows1)):
                    a1 = xt[pl.ds(si1 + h * LANES, LANES)]
                    a2 = xt[pl.ds(si2 + h * LANES, LANES)]
                    val = c0n + a1 * c1n + a2 * (c2n + a1 * c3n)
                    plsc.store_scatter(obuf, [rows, ncol], val)

        pltpu.async_copy(
            obuf, out_hbm.at[pl.ds(b0, B_W), pl.ds(n0, NC_OUT)], osem)

    _chunk(0, ob0, sem_o0, True)
    _chunk(1, ob1, sem_o1, True)

    @pl.loop(1, T_STEPS // 2)
    def _pair(tp):
        _chunk(2 * tp, ob0, sem_o0, False)
        _chunk(2 * tp + 1, ob1, sem_o1, False)

    pltpu.make_async_copy(
        ob0, out_hbm.at[pl.ds(b0, B_W), pl.ds(0, NC_OUT)], sem_o0).wait()
    pltpu.make_async_copy(
        ob1, out_hbm.at[pl.ds(b0, B_W), pl.ds(0, NC_OUT)], sem_o1).wait()


@functools.partial(
    pl.kernel,
    out_type=jax.ShapeDtypeStruct((BATCH, NUM_N), jnp.float32),
    mesh=plsc.VectorSubcoreMesh(core_axis_name="c", subcore_axis_name="s"),
    compiler_params=pltpu.CompilerParams(
        needs_layout_passes=False, disable_bounds_checks=True),
    scratch_types=[
        pltpu.VMEM((IN_SZ * BP,), jnp.float32),
        pltpu.VMEM((B_W * CCH,), jnp.float32),
        pltpu.VMEM((5 * NUM_N,), jnp.float32),
        pltpu.VMEM((B_W, NC_OUT), jnp.float32),
        pltpu.VMEM((B_W, NC_OUT), jnp.float32),
        pltpu.SemaphoreType.DMA,
        pltpu.SemaphoreType.DMA,
        pltpu.SemaphoreType.DMA,
        pltpu.SemaphoreType.DMA,
    ],
)
def _sc_combine(*refs):
    _sc_body(*refs)


def kernel(x, w, conn_indices):
    pk = _packed_side(w, conn_indices)
    return _sc_combine(x, pk)


def run():
    import reference as R
    d = R.setup_inputs(0)
    return kernel(d["x"], d["w"], d["conn_indices"])


# speedup vs baseline: 3.7106x; 3.7106x over previous
import functools

import jax
import jax.numpy as jnp
from jax import lax
from jax.experimental import pallas as pl
from jax.experimental.pallas import tpu as pltpu
from jax.experimental.pallas import tpu_sc as plsc

NUM_N = 8192
IN_SZ = 2048
BATCH = 1024

SC_CORES = 2
SC_SUBCORES = 16
NW = SC_CORES * SC_SUBCORES
LANES = 16
B_W = BATCH // NW
NC_OUT = 256
T_STEPS = NUM_N // NC_OUT


def _coef_body(wt_ref, it_ref, pk_ref):
    wv = wt_ref[...]
    m = jnp.max(wv, axis=0, keepdims=True)
    e = jnp.exp(wv - m)
    p = e / jnp.sum(e, axis=0, keepdims=True)
    c0 = p[8] + p[9] + p[10] + p[11] + p[12] + p[13] + p[14] + p[15]
    c1 = p[2] + p[3] + p[6] + p[7] - p[8] - p[9] - p[12] - p[13]
    c2 = p[4] + p[5] + p[6] + p[7] - p[8] - p[9] - p[10] - p[11]
    c3 = (p[1] - p[2] - p[4] - 2.0 * p[6] - p[7] + p[8] + 2.0 * p[9]
          + p[11] + p[13] - p[14])
    i12 = it_ref[0, :] | (it_ref[1, :] << 16)
    pk_ref[pl.ds(0, NUM_N)] = lax.bitcast_convert_type(i12, jnp.float32)
    pk_ref[pl.ds(NUM_N, NUM_N)] = c0
    pk_ref[pl.ds(2 * NUM_N, NUM_N)] = c1
    pk_ref[pl.ds(3 * NUM_N, NUM_N)] = c2
    pk_ref[pl.ds(4 * NUM_N, NUM_N)] = c3


def _packed_side(w, conn_indices):
    return pl.pallas_call(
        _coef_body,
        out_shape=jax.ShapeDtypeStruct((5 * NUM_N,), jnp.float32),
    )(w.T, conn_indices.T)


def _flatten_body(x_ref, o_ref):
    o_ref[...] = x_ref[...].reshape(B_W * IN_SZ)


def _flatten_x(x):
    return pl.pallas_call(
        _flatten_body,
        grid=(NW,),
        in_specs=[pl.BlockSpec((B_W, IN_SZ), lambda w: (w, 0))],
        out_specs=pl.BlockSpec((B_W * IN_SZ,), lambda w: (w,)),
        out_shape=jax.ShapeDtypeStruct((BATCH * IN_SZ,), jnp.float32),
    )(x)


def _sc_body(x_hbm, pk_hbm, out_hbm, xc, cb, ob0, ob1,
             sem_x, sem_pk, sem_o0, sem_o1):
    wid = lax.axis_index("s") * SC_CORES + lax.axis_index("c")
    b0 = wid * B_W
    for r in range(B_W):
        pltpu.async_copy(
            x_hbm.at[b0 + r], xc.at[pl.ds(r * IN_SZ, IN_SZ)], sem_x)
    cp = pltpu.async_copy(pk_hbm, cb, sem_pk)
    for r in range(B_W):
        pltpu.make_async_copy(
            x_hbm.at[b0 + r], xc.at[pl.ds(r * IN_SZ, IN_SZ)], sem_x).wait()
    cp.wait()

    def _chunk(t, obuf, osem, first):
        n0 = t * NC_OUT
        if not first:
            pltpu.make_async_copy(
                obuf, out_hbm.at[pl.ds(b0, B_W), pl.ds(n0, NC_OUT)],
                osem).wait()

        @pl.loop(0, NC_OUT // LANES)
        def _group(g):
            gb = n0 + g * LANES
            i12 = plsc.bitcast(cb[pl.ds(gb, LANES)], jnp.int32)
            i1g = i12 & 0xFFFF
            i2g = lax.shift_right_logical(i12, 16)
            c0g = cb[pl.ds(NUM_N + gb, LANES)]
            c1g = cb[pl.ds(2 * NUM_N + gb, LANES)]
            c2g = cb[pl.ds(3 * NUM_N + gb, LANES)]
            c3g = cb[pl.ds(4 * NUM_N + gb, LANES)]
            go = g * LANES

            @plsc.parallel_loop(0, B_W, 1, unroll=4)
            def _row(b):
                base = b * IN_SZ
                a1 = plsc.load_gather(xc, [i1g + base])
                a2 = plsc.load_gather(xc, [i2g + base])
                obuf[b, pl.ds(go, LANES)] = (
                    c0g + a1 * c1g + a2 * (c2g + a1 * c3g))

        pltpu.async_copy(
            obuf, out_hbm.at[pl.ds(b0, B_W), pl.ds(n0, NC_OUT)], osem)

    _chunk(0, ob0, sem_o0, True)
    _chunk(1, ob1, sem_o1, True)

    @pl.loop(1, T_STEPS // 2)
    def _pair(tp):
        _chunk(2 * tp, ob0, sem_o0, False)
        _chunk(2 * tp + 1, ob1, sem_o1, False)

    pltpu.make_async_copy(
        ob0, out_hbm.at[pl.ds(b0, B_W), pl.ds(0, NC_OUT)], sem_o0).wait()
    pltpu.make_async_copy(
        ob1, out_hbm.at[pl.ds(b0, B_W), pl.ds(0, NC_OUT)], sem_o1).wait()


@functools.partial(
    pl.kernel,
    out_type=jax.ShapeDtypeStruct((BATCH, NUM_N), jnp.float32),
    mesh=plsc.VectorSubcoreMesh(core_axis_name="c", subcore_axis_name="s"),
    compiler_params=pltpu.CompilerParams(
        needs_layout_passes=False, disable_bounds_checks=True,
        skip_device_barrier=True),
    scratch_types=[
        pltpu.VMEM((B_W * IN_SZ,), jnp.float32),
        pltpu.VMEM((5 * NUM_N,), jnp.float32),
        pltpu.VMEM((B_W, NC_OUT), jnp.float32),
        pltpu.VMEM((B_W, NC_OUT), jnp.float32),
        pltpu.SemaphoreType.DMA,
        pltpu.SemaphoreType.DMA,
        pltpu.SemaphoreType.DMA,
        pltpu.SemaphoreType.DMA,
    ],
)
def _sc_combine(*refs):
    _sc_body(*refs)


def kernel(x, w, conn_indices):
    pk = _packed_side(w, conn_indices)
    return _sc_combine(x, pk)
